# Initial kernel scaffold; baseline (speedup 1.0000x reference)
import jax
import jax.numpy as jnp
from jax.experimental import pallas as pl


def kernel(x, neigh, table):
    raise NotImplementedError("write your pallas kernel here")


# speedup vs baseline: 3.1184x; 3.1184x over previous
import numpy as np
import jax
import jax.numpy as jnp
from jax import lax
from jax.experimental import pallas as pl
from jax.experimental.pallas import tpu as pltpu
from jax.experimental.pallas import tpu_sc as plsc

_D = 64
_B = 4096
_L = 200
_ROWS = _B * _L
_NW = 32
_PER_W = _ROWS // _NW
_CHUNK = 200
_NCH = _PER_W // _CHUNK
_IDXW = 100


def _pe_table():
    pos = jnp.arange(_L, dtype=jnp.float32)[:, None]
    div = jnp.exp(jnp.arange(0, _D, 2, dtype=jnp.float32) * (-(np.log(10000.0) / _D)))
    angles = pos * div[None, :]
    pe = jnp.zeros((_L, _D), dtype=jnp.float32)
    pe = pe.at[:, 0::2].set(jnp.sin(angles))
    pe = pe.at[:, 1::2].set(jnp.cos(angles))
    return pe


def _sc_body(x_hbm, pe_hbm, tab_hbm, out_hbm, pe_v, idx_v, buf_v, gsem):
    wid = lax.axis_index("s") * 2 + lax.axis_index("c")
    pltpu.sync_copy(pe_hbm, pe_v)
    base_ix_row = wid * (_PER_W // _IDXW)
    base_out_row = wid * _PER_W

    def chunk_body(c, carry):
        ix_row = base_ix_row + c * 2
        out_row = base_out_row + c * _CHUNK
        pltpu.sync_copy(x_hbm.at[pl.ds(ix_row, 2)], idx_v)
        cp0 = pltpu.async_copy(tab_hbm.at[idx_v.at[0]], buf_v.at[pl.ds(0, _IDXW)], gsem)
        cp1 = pltpu.async_copy(tab_hbm.at[idx_v.at[1]], buf_v.at[pl.ds(_IDXW, _IDXW)], gsem)
        cp0.wait()
        cp1.wait()

        def add_row(r, carry2):
            for g in range(4):
                sl = pl.ds(g * 16, 16)
                buf_v[r, sl] += pe_v[r, sl]
            return carry2

        lax.fori_loop(0, _CHUNK, add_row, 0)
        pltpu.sync_copy(buf_v, out_hbm.at[pl.ds(out_row, _CHUNK)])
        return carry

    lax.fori_loop(0, _NCH, chunk_body, 0)


def kernel(x, neigh, table):
    del neigh
    x2 = x.reshape(_ROWS // _IDXW, _IDXW).astype(jnp.int32)
    pe = _pe_table()
    mesh = plsc.VectorSubcoreMesh(core_axis_name="c", subcore_axis_name="s")
    run = pl.kernel(
        _sc_body,
        out_type=jax.ShapeDtypeStruct((_ROWS, _D), jnp.float32),
        mesh=mesh,
        compiler_params=pltpu.CompilerParams(use_tc_tiling_on_sc=False),
        scratch_types=[
            pltpu.VMEM((_L, _D), jnp.float32),
            pltpu.VMEM((2, _IDXW), jnp.int32),
            pltpu.VMEM((_CHUNK, _D), jnp.float32),
            pltpu.SemaphoreType.DMA,
        ],
    )
    out = run(x2, pe, table)
    return out.reshape(_B, _L, _D)


# speedup vs baseline: 4.2381x; 1.3590x over previous
import numpy as np
import jax
import jax.numpy as jnp
from jax import lax
from jax.experimental import pallas as pl
from jax.experimental.pallas import tpu as pltpu
from jax.experimental.pallas import tpu_sc as plsc

_D = 64
_B = 4096
_L = 200
_ROWS = _B * _L
_NW = 32
_PER_W = _ROWS // _NW
_CHUNK = 200
_NCH = _PER_W // _CHUNK
_IDXW = 100
_IDXROWS = _PER_W // _IDXW
_NBUF = 4


def _pe_table():
    pos = jnp.arange(_L, dtype=jnp.float32)[:, None]
    div = jnp.exp(jnp.arange(0, _D, 2, dtype=jnp.float32) * (-(np.log(10000.0) / _D)))
    angles = pos * div[None, :]
    pe = jnp.zeros((_L, _D), dtype=jnp.float32)
    pe = pe.at[:, 0::2].set(jnp.sin(angles))
    pe = pe.at[:, 1::2].set(jnp.cos(angles))
    return pe


def _sc_body(x_hbm, pe_hbm, tab_hbm, out_hbm, pe_v, idx_v, buf_v, *sems):
    gsems = sems[:_NBUF]
    osems = sems[_NBUF:]
    wid = lax.axis_index("s") * 2 + lax.axis_index("c")
    pltpu.sync_copy(pe_hbm, pe_v)
    pltpu.sync_copy(x_hbm.at[pl.ds(wid * _IDXROWS, _IDXROWS)], idx_v)
    base_out = wid * _PER_W

    def gather(c, b, start):
        r = 2 * c
        for j in range(2):
            cp = pltpu.make_async_copy(
                tab_hbm.at[idx_v.at[r + j]],
                buf_v.at[b, pl.ds(j * _IDXW, _IDXW)],
                gsems[b],
            )
            if start:
                cp.start()
            else:
                cp.wait()

    def drain_out(b):
        pltpu.make_async_copy(
            buf_v.at[b], out_hbm.at[pl.ds(base_out, _CHUNK)], osems[b]
        ).wait()

    gather(0, 0, start=True)
    gather(1, 1, start=True)

    def step(cc, carry):
        for b in range(_NBUF):
            c = _NBUF * cc + b
            gather(c, b, start=False)
            b2 = (b + 2) % _NBUF
            if b < 2:
                @pl.when(cc > 0)
                def _():
                    drain_out(b2)
                gather(c + 2, b2, start=True)
            else:
                @pl.when(cc < _NCH // _NBUF - 1)
                def _():
                    drain_out(b2)
                    gather(c + 2, b2, start=True)

            @plsc.parallel_loop(0, _CHUNK, unroll=4)
            def _(r):
                for g in range(4):
                    sl = pl.ds(g * 16, 16)
                    plsc.addupdate(buf_v.at[b, r, sl], pe_v[r, sl])

            pltpu.async_copy(
                buf_v.at[b],
                out_hbm.at[pl.ds(base_out + c * _CHUNK, _CHUNK)],
                osems[b],
            )
        return carry

    lax.fori_loop(0, _NCH // _NBUF, step, 0)
    for b in range(_NBUF):
        drain_out(b)


def kernel(x, neigh, table):
    del neigh
    x2 = x.reshape(_ROWS // _IDXW, _IDXW).astype(jnp.int32)
    pe = _pe_table()
    mesh = plsc.VectorSubcoreMesh(core_axis_name="c", subcore_axis_name="s")
    run = pl.kernel(
        _sc_body,
        out_type=jax.ShapeDtypeStruct((_ROWS, _D), jnp.float32),
        mesh=mesh,
        compiler_params=pltpu.CompilerParams(use_tc_tiling_on_sc=False),
        scratch_types=[
            pltpu.VMEM((_L, _D), jnp.float32),
            pltpu.VMEM((_IDXROWS, _IDXW), jnp.int32),
            pltpu.VMEM((_NBUF, _CHUNK, _D), jnp.float32),
        ]
        + [pltpu.SemaphoreType.DMA] * (2 * _NBUF),
    )
    out = run(x2, pe, table)
    return out.reshape(_B, _L, _D)
